# Initial kernel scaffold; baseline (speedup 1.0000x reference)
import jax
import jax.numpy as jnp
from jax.experimental import pallas as pl


def kernel(left_image, right_image, chr_idx, W_embed, b_embed, W_logits, b_logits, prototypes):
    raise NotImplementedError("write your pallas kernel here")


# speedup vs baseline: 1.9996x; 1.9996x over previous
import functools

import jax
import jax.numpy as jnp
from jax.experimental import pallas as pl
from jax.experimental.pallas import tpu as pltpu

B = 4096
HW = 128
D = 128
T = 24
K = 8
FEAT = 512
BS = 128


def _pool_feat(img, bs):
    x = img.reshape(bs * 16, 8, HW).sum(axis=1) * (1.0 / 64.0)
    r = jax.lax.broadcasted_iota(jnp.int32, (HW, 16), 0) // 8
    c = jax.lax.broadcasted_iota(jnp.int32, (HW, 16), 1)
    pc = (r == c).astype(jnp.float32)
    p = jnp.dot(x, pc, preferred_element_type=jnp.float32, precision=jax.lax.Precision.HIGHEST)
    p3 = p.reshape(bs, 16, 16)
    return jnp.concatenate([p3[:, i, :] for i in range(16)], axis=1)


def _body(l_ref, r_ref, chr_ref, we_ref, be_ref, wl_ref, bl_ref, p_ref,
          logits_o, emb_o, dists_o, mind_o, midx_o, *, bs):
    fl = _pool_feat(l_ref[...], bs)
    fr = _pool_feat(r_ref[...], bs)
    feat = jnp.concatenate([fl, fr], axis=1)

    emb = jnp.dot(feat, we_ref[...], preferred_element_type=jnp.float32) + be_ref[...]
    emb_o[...] = emb
    logits_o[...] = jnp.dot(emb, wl_ref[...], preferred_element_type=jnp.float32) + bl_ref[...]

    n1 = jnp.sqrt(jnp.sum(emb * emb, axis=1, keepdims=True))
    emb_n = emb / jnp.maximum(n1, 1e-12)
    na = jnp.maximum(jnp.sqrt(jnp.sum(emb_n * emb_n, axis=1, keepdims=True)), 1e-8)
    emb_s = emb_n / na

    p0 = p_ref[...]
    pn1 = jnp.sqrt(jnp.sum(p0 * p0, axis=1, keepdims=True))
    pn = p0 / jnp.maximum(pn1, 1e-12)
    nb = jnp.maximum(jnp.sqrt(jnp.sum(pn * pn, axis=1, keepdims=True)), 1e-8)
    ps = pn / nb

    sims = jax.lax.dot_general(emb_s, ps, (((1,), (1,)), ((), ())),
                               preferred_element_type=jnp.float32,
                               precision=jax.lax.Precision.HIGHEST)
    d_all = 1.0 - sims

    col = jax.lax.broadcasted_iota(jnp.int32, (bs, T * K), 1)
    chrv = chr_ref[...][:, 0:1]
    mask = (col // K) == chrv
    dmask = jnp.where(mask, d_all, 0.0)
    er = jax.lax.broadcasted_iota(jnp.int32, (T * K, K), 0) % K
    ec = jax.lax.broadcasted_iota(jnp.int32, (T * K, K), 1)
    e = (er == ec).astype(jnp.float32)
    dists = jnp.dot(dmask, e, preferred_element_type=jnp.float32, precision=jax.lax.Precision.HIGHEST)
    dists_o[...] = dists

    mind = jnp.min(dists, axis=1, keepdims=True)
    mind_o[...] = jnp.broadcast_to(mind, (bs, K))
    kio = jax.lax.broadcasted_iota(jnp.int32, (bs, K), 1)
    midx = jnp.min(jnp.where(dists == mind, kio, K), axis=1, keepdims=True)
    midx_o[...] = jnp.broadcast_to(midx, (bs, K))


@jax.jit
def kernel(left_image, right_image, chr_idx, W_embed, b_embed, W_logits, b_logits, prototypes):
    bs = BS
    grid = (B // bs,)
    chrb = jnp.broadcast_to(chr_idx.astype(jnp.int32)[:, None], (B, 128))
    p2 = prototypes.reshape(T * K, D)

    outs = pl.pallas_call(
        functools.partial(_body, bs=bs),
        grid=grid,
        in_specs=[
            pl.BlockSpec((bs, 1, HW, HW), lambda i: (i, 0, 0, 0)),
            pl.BlockSpec((bs, 1, HW, HW), lambda i: (i, 0, 0, 0)),
            pl.BlockSpec((bs, 128), lambda i: (i, 0)),
            pl.BlockSpec((FEAT, D), lambda i: (0, 0)),
            pl.BlockSpec((1, D), lambda i: (0, 0)),
            pl.BlockSpec((D, T), lambda i: (0, 0)),
            pl.BlockSpec((1, T), lambda i: (0, 0)),
            pl.BlockSpec((T * K, D), lambda i: (0, 0)),
        ],
        out_specs=[
            pl.BlockSpec((bs, T), lambda i: (i, 0)),
            pl.BlockSpec((bs, D), lambda i: (i, 0)),
            pl.BlockSpec((bs, K), lambda i: (i, 0)),
            pl.BlockSpec((bs, K), lambda i: (i, 0)),
            pl.BlockSpec((bs, K), lambda i: (i, 0)),
        ],
        out_shape=[
            jax.ShapeDtypeStruct((B, T), jnp.float32),
            jax.ShapeDtypeStruct((B, D), jnp.float32),
            jax.ShapeDtypeStruct((B, K), jnp.float32),
            jax.ShapeDtypeStruct((B, K), jnp.float32),
            jax.ShapeDtypeStruct((B, K), jnp.int32),
        ],
    )(left_image, right_image, chrb, W_embed, b_embed.reshape(1, D),
      W_logits, b_logits.reshape(1, T), p2)

    logits, emb, dists, mind, midx = outs
    return (logits, emb, dists, mind[:, 0], midx[:, 0], prototypes)


# speedup vs baseline: 2.6575x; 1.3290x over previous
import functools

import jax
import jax.numpy as jnp
from jax.experimental import pallas as pl
from jax.experimental.pallas import tpu as pltpu

B = 4096
HW = 128
D = 128
T = 24
K = 8
FEAT = 512
BS = 128


def _pool_feat(img_ref, bs):
    parts = [img_ref[:, 0, pl.Slice(k, 16, 8), :] for k in range(8)]
    acc = parts[0]
    for pk in parts[1:]:
        acc = acc + pk
    x = acc.reshape(bs * 16, HW) * (1.0 / 64.0)
    r = jax.lax.broadcasted_iota(jnp.int32, (HW, 16), 0) // 8
    c = jax.lax.broadcasted_iota(jnp.int32, (HW, 16), 1)
    pc = (r == c).astype(jnp.float32)
    p = jnp.dot(x, pc, preferred_element_type=jnp.float32, precision=jax.lax.Precision.HIGHEST)
    p3 = p.reshape(bs, 16, 16)
    return jnp.concatenate([p3[:, i, :] for i in range(16)], axis=1)


def _body(l_ref, r_ref, chr_ref, we_ref, be_ref, wl_ref, bl_ref, p_ref,
          logits_o, emb_o, dists_o, mind_o, midx_o, *, bs):
    fl = _pool_feat(l_ref, bs)
    fr = _pool_feat(r_ref, bs)
    feat = jnp.concatenate([fl, fr], axis=1)

    emb = jnp.dot(feat, we_ref[...], preferred_element_type=jnp.float32) + be_ref[...]
    emb_o[...] = emb
    logits_o[...] = jnp.dot(emb, wl_ref[...], preferred_element_type=jnp.float32) + bl_ref[...]

    n1 = jnp.sqrt(jnp.sum(emb * emb, axis=1, keepdims=True))
    emb_n = emb / jnp.maximum(n1, 1e-12)
    na = jnp.maximum(jnp.sqrt(jnp.sum(emb_n * emb_n, axis=1, keepdims=True)), 1e-8)
    emb_s = emb_n / na

    p0 = p_ref[...]
    pn1 = jnp.sqrt(jnp.sum(p0 * p0, axis=1, keepdims=True))
    pn = p0 / jnp.maximum(pn1, 1e-12)
    nb = jnp.maximum(jnp.sqrt(jnp.sum(pn * pn, axis=1, keepdims=True)), 1e-8)
    ps = pn / nb

    sims = jax.lax.dot_general(emb_s, ps, (((1,), (1,)), ((), ())),
                               preferred_element_type=jnp.float32,
                               precision=jax.lax.Precision.HIGHEST)
    d_all = 1.0 - sims

    col = jax.lax.broadcasted_iota(jnp.int32, (bs, T * K), 1)
    chrv = chr_ref[...][:, 0:1]
    mask = (col // K) == chrv
    dmask = jnp.where(mask, d_all, 0.0)
    er = jax.lax.broadcasted_iota(jnp.int32, (T * K, K), 0) % K
    ec = jax.lax.broadcasted_iota(jnp.int32, (T * K, K), 1)
    e = (er == ec).astype(jnp.float32)
    dists = jnp.dot(dmask, e, preferred_element_type=jnp.float32, precision=jax.lax.Precision.HIGHEST)
    dists_o[...] = dists

    mind = jnp.min(dists, axis=1, keepdims=True)
    mind_o[...] = jnp.broadcast_to(mind, (bs, K))
    kio = jax.lax.broadcasted_iota(jnp.int32, (bs, K), 1)
    midx = jnp.min(jnp.where(dists == mind, kio, K), axis=1, keepdims=True)
    midx_o[...] = jnp.broadcast_to(midx, (bs, K))


@jax.jit
def kernel(left_image, right_image, chr_idx, W_embed, b_embed, W_logits, b_logits, prototypes):
    bs = BS
    grid = (B // bs,)
    chrb = jnp.broadcast_to(chr_idx.astype(jnp.int32)[:, None], (B, 128))
    p2 = prototypes.reshape(T * K, D)

    outs = pl.pallas_call(
        functools.partial(_body, bs=bs),
        grid=grid,
        in_specs=[
            pl.BlockSpec((bs, 1, HW, HW), lambda i: (i, 0, 0, 0)),
            pl.BlockSpec((bs, 1, HW, HW), lambda i: (i, 0, 0, 0)),
            pl.BlockSpec((bs, 128), lambda i: (i, 0)),
            pl.BlockSpec((FEAT, D), lambda i: (0, 0)),
            pl.BlockSpec((1, D), lambda i: (0, 0)),
            pl.BlockSpec((D, T), lambda i: (0, 0)),
            pl.BlockSpec((1, T), lambda i: (0, 0)),
            pl.BlockSpec((T * K, D), lambda i: (0, 0)),
        ],
        out_specs=[
            pl.BlockSpec((bs, T), lambda i: (i, 0)),
            pl.BlockSpec((bs, D), lambda i: (i, 0)),
            pl.BlockSpec((bs, K), lambda i: (i, 0)),
            pl.BlockSpec((bs, K), lambda i: (i, 0)),
            pl.BlockSpec((bs, K), lambda i: (i, 0)),
        ],
        out_shape=[
            jax.ShapeDtypeStruct((B, T), jnp.float32),
            jax.ShapeDtypeStruct((B, D), jnp.float32),
            jax.ShapeDtypeStruct((B, K), jnp.float32),
            jax.ShapeDtypeStruct((B, K), jnp.float32),
            jax.ShapeDtypeStruct((B, K), jnp.int32),
        ],
    )(left_image, right_image, chrb, W_embed, b_embed.reshape(1, D),
      W_logits, b_logits.reshape(1, T), p2)

    logits, emb, dists, mind, midx = outs
    return (logits, emb, dists, mind[:, 0], midx[:, 0], prototypes)
